# Initial kernel scaffold; baseline (speedup 1.0000x reference)
import functools

import jax
import jax.numpy as jnp
from jax import lax
from jax.experimental import pallas as pl
from jax.experimental.pallas import tpu as pltpu
from jax.experimental.pallas import tpu_sc as plsc

N_NODES = 10000
N_PAD = 10240
D = 128
N_EDGES = 320000
NC = 2
NS = 16
NW = NC * NS
EPW = N_EDGES // NW
K = 80
CH = EPW // K
ZR = N_PAD // NS
BLK = 500
GRID = N_NODES // BLK

_mesh = plsc.VectorSubcoreMesh(core_axis_name="c", subcore_axis_name="s")


def _sc_hist(col3, zeros, ones):
  @functools.partial(
      pl.kernel,
      out_type=jax.ShapeDtypeStruct((NC, N_PAD, 16), jnp.float32),
      mesh=_mesh,
      scratch_types=[
          pltpu.VMEM((CH, K), jnp.int32),
          pltpu.VMEM((K, 16), jnp.float32),
          pltpu.VMEM_SHARED((N_PAD, 16), jnp.float32),
      ],
  )
  def hist(col_hbm, z_hbm, ones_hbm, out_hbm, colv, onesv, acc):
    cid = lax.axis_index("c")
    sid = lax.axis_index("s")
    wid = cid * NS + sid
    pltpu.sync_copy(col_hbm.at[wid], colv)
    pltpu.sync_copy(ones_hbm, onesv)
    pltpu.sync_copy(z_hbm, acc.at[pl.ds(sid * ZR, ZR)])
    plsc.subcore_barrier()

    @pl.loop(0, CH)
    def _(c):
      pltpu.sync_copy(onesv, acc.at[colv.at[c]], add=True)

    plsc.subcore_barrier()
    pltpu.sync_copy(acc.at[pl.ds(sid * ZR, ZR)],
                    out_hbm.at[cid].at[pl.ds(sid * ZR, ZR)])

  return hist(col3, zeros, ones)


def _sc_gather_scatter(g, row2, col3, zeros):
  @functools.partial(
      pl.kernel,
      out_type=jax.ShapeDtypeStruct((NC, N_PAD, D), jnp.float32),
      mesh=_mesh,
      scratch_types=[
          pltpu.VMEM((EPW,), jnp.int32),
          pltpu.VMEM((CH, K), jnp.int32),
          pltpu.VMEM((K, D), jnp.float32),
          pltpu.VMEM_SHARED((N_PAD, D), jnp.float32),
          pltpu.SemaphoreType.DMA,
      ],
  )
  def gs(g_hbm, row_hbm, col_hbm, z_hbm, out_hbm, rowv, colv, vals, acc, sem):
    cid = lax.axis_index("c")
    sid = lax.axis_index("s")
    wid = cid * NS + sid
    pltpu.sync_copy(row_hbm.at[wid], rowv)
    pltpu.sync_copy(col_hbm.at[wid], colv)
    pltpu.sync_copy(z_hbm, acc.at[pl.ds(sid * ZR, ZR)])
    plsc.subcore_barrier()

    @pl.loop(0, CH)
    def _(c):
      pltpu.async_copy(g_hbm.at[rowv.at[pl.ds(c * K, K)]], vals, sem).wait()
      pltpu.sync_copy(vals, acc.at[colv.at[c]], add=True)

    plsc.subcore_barrier()
    pltpu.sync_copy(acc.at[pl.ds(sid * ZR, ZR)],
                    out_hbm.at[cid].at[pl.ds(sid * ZR, ZR)])

  return gs(g, row2, col3, zeros)


def _dinv(da_ref, db_ref):
  deg = 1.0 + da_ref[0, :, 0:1] + db_ref[0, :, 0:1]
  return lax.rsqrt(deg)


def _mm_scale_body(x_ref, w_ref, da_ref, db_ref, g_ref):
  h = jnp.dot(x_ref[...], w_ref[...], preferred_element_type=jnp.float32)
  g_ref[...] = h * _dinv(da_ref, db_ref)


def _mid_body(aa_ref, ab_ref, g_ref, da_ref, db_ref, b_ref, a_ref, w_ref,
              out_ref):
  dinv = _dinv(da_ref, db_ref)
  s = (aa_ref[0] + ab_ref[0] + g_ref[...]) * dinv + b_ref[...]
  x1 = jnp.where(s >= 0, s, a_ref[...] * s)
  h = jnp.dot(x1, w_ref[...], preferred_element_type=jnp.float32)
  out_ref[...] = h * dinv


def _final_body(aa_ref, ab_ref, g_ref, da_ref, db_ref, b_ref, a_ref, out_ref):
  dinv = _dinv(da_ref, db_ref)
  s = (aa_ref[0] + ab_ref[0] + g_ref[...]) * dinv + b_ref[...]
  out_ref[...] = jnp.where(s >= 0, s, a_ref[...] * s)


def _acc_spec(core):
  return pl.BlockSpec((1, BLK, D), lambda i, c=core: (c, i, 0))


def _deg_spec(core):
  return pl.BlockSpec((1, BLK, 16), lambda i, c=core: (c, i, 0))


_row_spec = pl.BlockSpec((BLK, D), lambda i: (i, 0))
_w_spec = pl.BlockSpec((D, D), lambda i: (0, 0))
_vec_spec = pl.BlockSpec((1, D), lambda i: (0, 0))
_out_f32 = jax.ShapeDtypeStruct((N_NODES, D), jnp.float32)


def _tc_mm_scale(x, W, deg):
  return pl.pallas_call(
      _mm_scale_body,
      grid=(GRID,),
      in_specs=[_row_spec, _w_spec, _deg_spec(0), _deg_spec(1)],
      out_specs=_row_spec,
      out_shape=_out_f32,
  )(x, W, deg, deg)


def _tc_mid(acc, g, deg, b, a, W):
  return pl.pallas_call(
      _mid_body,
      grid=(GRID,),
      in_specs=[_acc_spec(0), _acc_spec(1), _row_spec, _deg_spec(0),
                _deg_spec(1), _vec_spec, _vec_spec, _w_spec],
      out_specs=_row_spec,
      out_shape=_out_f32,
  )(acc, acc, g, deg, deg, b, a, W)


def _tc_final(acc, g, deg, b, a):
  return pl.pallas_call(
      _final_body,
      grid=(GRID,),
      in_specs=[_acc_spec(0), _acc_spec(1), _row_spec, _deg_spec(0),
                _deg_spec(1), _vec_spec, _vec_spec],
      out_specs=_row_spec,
      out_shape=_out_f32,
  )(acc, acc, g, deg, deg, b, a)


def kernel(x, edge_index, W1, b1, W2, b2, a):
  row2 = edge_index[0].astype(jnp.int32).reshape(NW, EPW)
  col3 = edge_index[1].astype(jnp.int32).reshape(NW, CH, K)
  zeros_d = jnp.zeros((ZR, D), jnp.float32)
  zeros16 = jnp.zeros((ZR, 16), jnp.float32)
  ones16 = jnp.ones((K, 16), jnp.float32)
  b1r = b1.reshape(1, D)
  b2r = b2.reshape(1, D)
  ar = a.reshape(1, D)

  deg = _sc_hist(col3, zeros16, ones16)
  g1 = _tc_mm_scale(x, W1, deg)
  acc1 = _sc_gather_scatter(g1, row2, col3, zeros_d)
  g2 = _tc_mid(acc1, g1, deg, b1r, ar, W2)
  acc2 = _sc_gather_scatter(g2, row2, col3, zeros_d)
  return _tc_final(acc2, g2, deg, b2r, ar)


# speedup vs baseline: 17.5356x; 17.5356x over previous
import functools

import jax
import jax.numpy as jnp
from jax import lax
from jax.experimental import pallas as pl
from jax.experimental.pallas import tpu as pltpu
from jax.experimental.pallas import tpu_sc as plsc

N_NODES = 10000
N_PAD = 10240
D = 128
N_EDGES = 320000
NC = 2
NS = 16
NW = NC * NS
EPW = N_EDGES // NW
K = 80
CH = EPW // K
ZR = N_PAD // NS
BLK = 400
GRID = N_NODES // BLK

def _mesh():
  return plsc.VectorSubcoreMesh(
      core_axis_name="c", subcore_axis_name="s", num_cores=NC, num_subcores=NS)


def _sc_hist(col3, zeros, ones):
  @functools.partial(
      pl.kernel,
      out_type=jax.ShapeDtypeStruct((NC, N_PAD, D), jnp.float32),
      mesh=_mesh(),
      scratch_types=[
          pltpu.VMEM((CH, K), jnp.int32),
          pltpu.VMEM((K, D), jnp.float32),
          pltpu.VMEM_SHARED((N_PAD, D), jnp.float32),
      ],
  )
  def hist(col_hbm, z_hbm, ones_hbm, out_hbm, colv, onesv, acc):
    cid = lax.axis_index("c")
    sid = lax.axis_index("s")
    wid = cid * NS + sid
    pltpu.sync_copy(col_hbm.at[wid], colv)
    pltpu.sync_copy(ones_hbm, onesv)
    pltpu.sync_copy(z_hbm, acc.at[pl.ds(sid * ZR, ZR)])
    plsc.subcore_barrier()

    @pl.loop(0, CH)
    def _(c):
      pltpu.sync_copy(onesv, acc.at[colv.at[c]], add=True)

    plsc.subcore_barrier()
    pltpu.sync_copy(acc.at[pl.ds(sid * ZR, ZR)],
                    out_hbm.at[cid].at[pl.ds(sid * ZR, ZR)])

  return hist(col3, zeros, ones)


def _sc_gather_scatter(g, row2, col3, zeros):
  @functools.partial(
      pl.kernel,
      out_type=jax.ShapeDtypeStruct((NC, N_PAD, D), jnp.float32),
      mesh=_mesh(),
      scratch_types=[
          pltpu.VMEM((EPW,), jnp.int32),
          pltpu.VMEM((CH, K), jnp.int32),
          pltpu.VMEM((K, D), jnp.float32),
          pltpu.VMEM_SHARED((N_PAD, D), jnp.float32),
          pltpu.SemaphoreType.DMA,
      ],
  )
  def gs(g_hbm, row_hbm, col_hbm, z_hbm, out_hbm, rowv, colv, vals, acc, sem):
    cid = lax.axis_index("c")
    sid = lax.axis_index("s")
    wid = cid * NS + sid
    pltpu.sync_copy(row_hbm.at[wid], rowv)
    pltpu.sync_copy(col_hbm.at[wid], colv)
    pltpu.sync_copy(z_hbm, acc.at[pl.ds(sid * ZR, ZR)])
    plsc.subcore_barrier()

    @pl.loop(0, CH)
    def _(c):
      pltpu.async_copy(g_hbm.at[rowv.at[pl.ds(c * K, K)]], vals, sem).wait()
      pltpu.sync_copy(vals, acc.at[colv.at[c]], add=True)

    plsc.subcore_barrier()
    pltpu.sync_copy(acc.at[pl.ds(sid * ZR, ZR)],
                    out_hbm.at[cid].at[pl.ds(sid * ZR, ZR)])

  return gs(g, row2, col3, zeros)


def _dinv(da_ref, db_ref):
  deg = 1.0 + da_ref[0, :, 0:1] + db_ref[0, :, 0:1]
  return lax.rsqrt(deg)


def _mm_scale_body(x_ref, w_ref, da_ref, db_ref, g_ref):
  h = jnp.dot(x_ref[...], w_ref[...], preferred_element_type=jnp.float32)
  g_ref[...] = h * _dinv(da_ref, db_ref)


def _mid_body(aa_ref, ab_ref, g_ref, da_ref, db_ref, b_ref, a_ref, w_ref,
              out_ref):
  dinv = _dinv(da_ref, db_ref)
  s = (aa_ref[0] + ab_ref[0] + g_ref[...]) * dinv + b_ref[...]
  x1 = jnp.where(s >= 0, s, a_ref[...] * s)
  h = jnp.dot(x1, w_ref[...], preferred_element_type=jnp.float32)
  out_ref[...] = h * dinv


def _final_body(aa_ref, ab_ref, g_ref, da_ref, db_ref, b_ref, a_ref, out_ref):
  dinv = _dinv(da_ref, db_ref)
  s = (aa_ref[0] + ab_ref[0] + g_ref[...]) * dinv + b_ref[...]
  out_ref[...] = jnp.where(s >= 0, s, a_ref[...] * s)


def _acc_spec(core):
  return pl.BlockSpec((1, BLK, D), lambda i, c=core: (c, i, 0))


def _deg_spec(core):
  return pl.BlockSpec((1, BLK, D), lambda i, c=core: (c, i, 0))


_row_spec = pl.BlockSpec((BLK, D), lambda i: (i, 0))
_w_spec = pl.BlockSpec((D, D), lambda i: (0, 0))
_vec_spec = pl.BlockSpec((1, D), lambda i: (0, 0))
_out_f32 = jax.ShapeDtypeStruct((N_NODES, D), jnp.float32)


def _tc_mm_scale(x, W, deg):
  return pl.pallas_call(
      _mm_scale_body,
      grid=(GRID,),
      in_specs=[_row_spec, _w_spec, _deg_spec(0), _deg_spec(1)],
      out_specs=_row_spec,
      out_shape=_out_f32,
  )(x, W, deg, deg)


def _tc_mid(acc, g, deg, b, a, W):
  return pl.pallas_call(
      _mid_body,
      grid=(GRID,),
      in_specs=[_acc_spec(0), _acc_spec(1), _row_spec, _deg_spec(0),
                _deg_spec(1), _vec_spec, _vec_spec, _w_spec],
      out_specs=_row_spec,
      out_shape=_out_f32,
  )(acc, acc, g, deg, deg, b, a, W)


def _tc_final(acc, g, deg, b, a):
  return pl.pallas_call(
      _final_body,
      grid=(GRID,),
      in_specs=[_acc_spec(0), _acc_spec(1), _row_spec, _deg_spec(0),
                _deg_spec(1), _vec_spec, _vec_spec],
      out_specs=_row_spec,
      out_shape=_out_f32,
  )(acc, acc, g, deg, deg, b, a)


def kernel(x, edge_index, W1, b1, W2, b2, a):
  row2 = edge_index[0].astype(jnp.int32).reshape(NW, EPW)
  col3 = edge_index[1].astype(jnp.int32).reshape(NW, CH, K)
  zeros_d = jnp.zeros((ZR, D), jnp.float32)
  ones_d = jnp.ones((K, D), jnp.float32)
  b1r = b1.reshape(1, D)
  b2r = b2.reshape(1, D)
  ar = a.reshape(1, D)

  deg = _sc_hist(col3, zeros_d, ones_d)
  g1 = _tc_mm_scale(x, W1, deg)
  acc1 = _sc_gather_scatter(g1, row2, col3, zeros_d)
  g2 = _tc_mid(acc1, g1, deg, b1r, ar, W2)
  acc2 = _sc_gather_scatter(g2, row2, col3, zeros_d)
  return _tc_final(acc2, g2, deg, b2r, ar)


# speedup vs baseline: 25.1520x; 1.4343x over previous
import functools

import jax
import jax.numpy as jnp
from jax import lax
from jax.experimental import pallas as pl
from jax.experimental.pallas import tpu as pltpu
from jax.experimental.pallas import tpu_sc as plsc

N_NODES = 10000
N_PAD = 10240
D = 128
N_EDGES = 320000
NC = 2
NS = 16
NW = NC * NS
EPW = N_EDGES // NW
K = 80
CH = EPW // K
ZR = N_PAD // NS
BLK = 400
GRID = N_NODES // BLK

def _mesh():
  return plsc.VectorSubcoreMesh(
      core_axis_name="c", subcore_axis_name="s", num_cores=NC, num_subcores=NS)


def _sc_hist(col3, zeros, ones):
  @functools.partial(
      pl.kernel,
      out_type=jax.ShapeDtypeStruct((NC, N_PAD, D), jnp.float32),
      mesh=_mesh(),
      scratch_types=[
          pltpu.VMEM((CH, K), jnp.int32),
          pltpu.VMEM((K, D), jnp.float32),
          pltpu.VMEM_SHARED((N_PAD, D), jnp.float32),
      ],
  )
  def hist(col_hbm, z_hbm, ones_hbm, out_hbm, colv, onesv, acc):
    cid = lax.axis_index("c")
    sid = lax.axis_index("s")
    wid = cid * NS + sid
    pltpu.sync_copy(col_hbm.at[wid], colv)
    pltpu.sync_copy(ones_hbm, onesv)
    pltpu.sync_copy(z_hbm, acc.at[pl.ds(sid * ZR, ZR)])
    plsc.subcore_barrier()

    @pl.loop(0, CH)
    def _(c):
      pltpu.sync_copy(onesv, acc.at[colv.at[c]], add=True)

    plsc.subcore_barrier()
    pltpu.sync_copy(acc.at[pl.ds(sid * ZR, ZR)],
                    out_hbm.at[cid].at[pl.ds(sid * ZR, ZR)])

  return hist(col3, zeros, ones)


def _sc_gather_scatter(g, row2, col3, zeros):
  @functools.partial(
      pl.kernel,
      out_type=jax.ShapeDtypeStruct((NC, N_PAD, D), jnp.float32),
      mesh=_mesh(),
      scratch_types=[
          pltpu.VMEM((EPW,), jnp.int32),
          pltpu.VMEM((CH, K), jnp.int32),
          pltpu.VMEM((K, D), jnp.float32),
          pltpu.VMEM((K, D), jnp.float32),
          pltpu.VMEM_SHARED((N_PAD, D), jnp.float32),
          pltpu.SemaphoreType.DMA,
          pltpu.SemaphoreType.DMA,
      ],
  )
  def gs(g_hbm, row_hbm, col_hbm, z_hbm, out_hbm, rowv, colv, v0, v1, acc,
         sem0, sem1):
    cid = lax.axis_index("c")
    sid = lax.axis_index("s")
    wid = cid * NS + sid
    pltpu.sync_copy(row_hbm.at[wid], rowv)
    pltpu.sync_copy(col_hbm.at[wid], colv)
    pltpu.sync_copy(z_hbm, acc.at[pl.ds(sid * ZR, ZR)])
    plsc.subcore_barrier()

    def gather(c, buf, sem):
      return pltpu.async_copy(g_hbm.at[rowv.at[pl.ds(c * K, K)]], buf, sem)

    gather(0, v0, sem0)

    @pl.loop(0, CH - 1, step=2)
    def _(c):
      gather(c + 1, v1, sem1)
      pltpu.make_async_copy(g_hbm.at[rowv.at[pl.ds(c * K, K)]], v0,
                            sem0).wait()
      pltpu.sync_copy(v0, acc.at[colv.at[c]], add=True)
      gather(c + 2, v0, sem0)
      pltpu.make_async_copy(g_hbm.at[rowv.at[pl.ds((c + 1) * K, K)]], v1,
                            sem1).wait()
      pltpu.sync_copy(v1, acc.at[colv.at[c + 1]], add=True)

    pltpu.make_async_copy(g_hbm.at[rowv.at[pl.ds((CH - 1) * K, K)]], v0,
                          sem0).wait()
    pltpu.sync_copy(v0, acc.at[colv.at[CH - 1]], add=True)

    plsc.subcore_barrier()
    pltpu.sync_copy(acc.at[pl.ds(sid * ZR, ZR)],
                    out_hbm.at[cid].at[pl.ds(sid * ZR, ZR)])

  return gs(g, row2, col3, zeros)


def _dinv(da_ref, db_ref):
  deg = 1.0 + da_ref[0, :, 0:1] + db_ref[0, :, 0:1]
  return lax.rsqrt(deg)


def _mm_scale_body(x_ref, w_ref, da_ref, db_ref, g_ref):
  h = jnp.dot(x_ref[...], w_ref[...], preferred_element_type=jnp.float32)
  g_ref[...] = h * _dinv(da_ref, db_ref)


def _mid_body(aa_ref, ab_ref, g_ref, da_ref, db_ref, b_ref, a_ref, w_ref,
              out_ref):
  dinv = _dinv(da_ref, db_ref)
  s = (aa_ref[0] + ab_ref[0] + g_ref[...]) * dinv + b_ref[...]
  x1 = jnp.where(s >= 0, s, a_ref[...] * s)
  h = jnp.dot(x1, w_ref[...], preferred_element_type=jnp.float32)
  out_ref[...] = h * dinv


def _final_body(aa_ref, ab_ref, g_ref, da_ref, db_ref, b_ref, a_ref, out_ref):
  dinv = _dinv(da_ref, db_ref)
  s = (aa_ref[0] + ab_ref[0] + g_ref[...]) * dinv + b_ref[...]
  out_ref[...] = jnp.where(s >= 0, s, a_ref[...] * s)


def _acc_spec(core):
  return pl.BlockSpec((1, BLK, D), lambda i, c=core: (c, i, 0))


def _deg_spec(core):
  return pl.BlockSpec((1, BLK, D), lambda i, c=core: (c, i, 0))


_row_spec = pl.BlockSpec((BLK, D), lambda i: (i, 0))
_w_spec = pl.BlockSpec((D, D), lambda i: (0, 0))
_vec_spec = pl.BlockSpec((1, D), lambda i: (0, 0))
_out_f32 = jax.ShapeDtypeStruct((N_NODES, D), jnp.float32)


def _tc_mm_scale(x, W, deg):
  return pl.pallas_call(
      _mm_scale_body,
      grid=(GRID,),
      in_specs=[_row_spec, _w_spec, _deg_spec(0), _deg_spec(1)],
      out_specs=_row_spec,
      out_shape=_out_f32,
  )(x, W, deg, deg)


def _tc_mid(acc, g, deg, b, a, W):
  return pl.pallas_call(
      _mid_body,
      grid=(GRID,),
      in_specs=[_acc_spec(0), _acc_spec(1), _row_spec, _deg_spec(0),
                _deg_spec(1), _vec_spec, _vec_spec, _w_spec],
      out_specs=_row_spec,
      out_shape=_out_f32,
  )(acc, acc, g, deg, deg, b, a, W)


def _tc_final(acc, g, deg, b, a):
  return pl.pallas_call(
      _final_body,
      grid=(GRID,),
      in_specs=[_acc_spec(0), _acc_spec(1), _row_spec, _deg_spec(0),
                _deg_spec(1), _vec_spec, _vec_spec],
      out_specs=_row_spec,
      out_shape=_out_f32,
  )(acc, acc, g, deg, deg, b, a)


def kernel(x, edge_index, W1, b1, W2, b2, a):
  row2 = edge_index[0].astype(jnp.int32).reshape(NW, EPW)
  col3 = edge_index[1].astype(jnp.int32).reshape(NW, CH, K)
  zeros_d = jnp.zeros((ZR, D), jnp.float32)
  ones_d = jnp.ones((K, D), jnp.float32)
  b1r = b1.reshape(1, D)
  b2r = b2.reshape(1, D)
  ar = a.reshape(1, D)

  deg = _sc_hist(col3, zeros_d, ones_d)
  g1 = _tc_mm_scale(x, W1, deg)
  acc1 = _sc_gather_scatter(g1, row2, col3, zeros_d)
  g2 = _tc_mid(acc1, g1, deg, b1r, ar, W2)
  acc2 = _sc_gather_scatter(g2, row2, col3, zeros_d)
  return _tc_final(acc2, g2, deg, b2r, ar)
